# baseline (speedup 1.0000x reference)
import numpy as np
import jax
import jax.numpy as jnp
from jax.experimental import pallas as pl

T = 2048
DMODEL = 2048
QLORA = 1536
NH = 16
DH = 128
ROPE_HD = 64
TOPK = 512
EPS = 1e-06
RB = 128
HALF = (DH - ROPE_HD) // 2

SOFTMAX_SCALE = DH ** -0.5
WEIGHTS_SCALE = NH ** -0.5


def _hadamard_matrix(n):
    H = np.array([[1.0]], dtype=np.float32)
    while H.shape[0] < n:
        H = np.block([[H, H], [H, -H]]).astype(np.float32)
    return H

_HAD = jnp.asarray(_hadamard_matrix(DH) * (DH ** -0.5), dtype=jnp.float32)


def _layernorm(x, w, b, eps):
    mu = jnp.mean(x, axis=-1, keepdims=True)
    var = jnp.mean((x - mu) ** 2, axis=-1, keepdims=True)
    return (x - mu) / jnp.sqrt(var + eps) * w + b


def _apply_rope(x, cos, sin):
    half = cos.shape[-1]
    x1 = x[..., :half]
    x2 = x[..., half:]
    return jnp.concatenate([x1 * cos - x2 * sin, x2 * cos + x1 * sin], axis=-1)


def _topk_sort(v):
    R, N = v.shape
    lane = jax.lax.broadcasted_iota(jnp.int32, (R, N), 1)
    i = lane
    k = 2
    while k <= N:
        j = k // 2
        while j >= 1:
            maskj = (lane & j) == 0
            dir_desc = (lane & k) == 0
            pv = jnp.where(maskj, jnp.roll(v, -j, axis=1), jnp.roll(v, j, axis=1))
            pi = jnp.where(maskj, jnp.roll(i, -j, axis=1), jnp.roll(i, j, axis=1))
            self_gt = (v > pv) | ((v == pv) & (i < pi))
            keep = dir_desc ^ maskj ^ self_gt
            v = jnp.where(keep, v, pv)
            i = jnp.where(keep, i, pi)
            j //= 2
        k *= 2
    return v[:, :TOPK], i[:, :TOPK]


def _main_kernel(q2_ref, k_ref, w_ref, vals_ref, idx_ref):
    b = pl.program_id(0)
    q2 = q2_ref[...].reshape(RB * NH, DH)
    scores = jax.lax.dot_general(
        q2, k_ref[...], (((1,), (1,)), ((), ())),
        preferred_element_type=jnp.float32)
    s2 = jnp.maximum(scores, 0.0)
    w = w_ref[...]
    wtile = jnp.broadcast_to(w[:, None, :], (RB, RB, NH)).reshape(RB, RB * NH)
    rowt = jax.lax.broadcasted_iota(jnp.int32, (RB, RB * NH), 0)
    colt = jax.lax.broadcasted_iota(jnp.int32, (RB, RB * NH), 1) // NH
    W2 = jnp.where(rowt == colt, wtile, 0.0)
    lg = jnp.dot(W2, s2, preferred_element_type=jnp.float32)
    row = b * RB + jax.lax.broadcasted_iota(jnp.int32, (RB, T), 0)
    col = jax.lax.broadcasted_iota(jnp.int32, (RB, T), 1)
    lg = jnp.where(row >= col, lg, jnp.float32(-1e30))
    vals, idx = _topk_sort(lg)
    vals_ref[...] = vals
    idx_ref[...] = idx


def kernel(hidden_states, q_lora, positions, wq_b, wk, k_norm_w, k_norm_b,
           weights_proj, cos_sin_cache):
    rot_dim = DH - ROPE_HD
    q = (q_lora @ wq_b).reshape(T, NH, DH)
    k = _layernorm(hidden_states @ wk, k_norm_w, k_norm_b, EPS)
    cos = cos_sin_cache[positions, : rot_dim // 2]
    sin = cos_sin_cache[positions, rot_dim // 2:]
    q_pe = _apply_rope(q[:, :, :rot_dim], cos[:, None, :], sin[:, None, :])
    q = jnp.concatenate([q_pe, q[:, :, rot_dim:]], axis=-1)
    k_pe = _apply_rope(k[:, :rot_dim], cos, sin)
    k = jnp.concatenate([k_pe, k[:, rot_dim:]], axis=-1)
    q = q @ _HAD
    k = k @ _HAD
    w = (hidden_states @ weights_proj) * SOFTMAX_SCALE * WEIGHTS_SCALE

    vals, idx = pl.pallas_call(
        _main_kernel,
        grid=(T // RB,),
        in_specs=[pl.BlockSpec((RB, NH * DH), lambda b: (b, 0)),
                  pl.BlockSpec((T, DH), lambda b: (0, 0)),
                  pl.BlockSpec((RB, NH), lambda b: (b, 0))],
        out_specs=[pl.BlockSpec((RB, TOPK), lambda b: (b, 0)),
                   pl.BlockSpec((RB, TOPK), lambda b: (b, 0))],
        out_shape=[jax.ShapeDtypeStruct((T, TOPK), jnp.float32),
                   jax.ShapeDtypeStruct((T, TOPK), jnp.int32)],
    )(q.reshape(T, NH * DH), k, w)
    return vals, idx


# speedup vs baseline: 1.0368x; 1.0368x over previous
import numpy as np
import jax
import jax.numpy as jnp
from jax.experimental import pallas as pl

T = 2048
DMODEL = 2048
QLORA = 1536
NH = 16
DH = 128
ROPE_HD = 64
TOPK = 512
EPS = 1e-06
RB = 128
HALF = (DH - ROPE_HD) // 2

SOFTMAX_SCALE = DH ** -0.5
WEIGHTS_SCALE = NH ** -0.5


def _hadamard_matrix(n):
    H = np.array([[1.0]], dtype=np.float32)
    while H.shape[0] < n:
        H = np.block([[H, H], [H, -H]]).astype(np.float32)
    return H

_HAD = jnp.asarray(_hadamard_matrix(DH) * (DH ** -0.5), dtype=jnp.float32)


def _layernorm(x, w, b, eps):
    mu = jnp.mean(x, axis=-1, keepdims=True)
    var = jnp.mean((x - mu) ** 2, axis=-1, keepdims=True)
    return (x - mu) / jnp.sqrt(var + eps) * w + b


def _apply_rope(x, cos, sin):
    half = cos.shape[-1]
    x1 = x[..., :half]
    x2 = x[..., half:]
    return jnp.concatenate([x1 * cos - x2 * sin, x2 * cos + x1 * sin], axis=-1)


def _topk_sort(v):
    R, N = v.shape
    lane = jax.lax.broadcasted_iota(jnp.int32, (R, N), 1)
    i = lane
    k = 2
    while k <= N:
        j = k // 2
        while j >= 1:
            maskj = (lane & j) == 0
            dir_desc = (lane & k) == 0
            pv = jnp.where(maskj, jnp.roll(v, -j, axis=1), jnp.roll(v, j, axis=1))
            pi = jnp.where(maskj, jnp.roll(i, -j, axis=1), jnp.roll(i, j, axis=1))
            self_gt = (v > pv) | ((v == pv) & (i < pi))
            keep = dir_desc ^ maskj ^ self_gt
            v = jnp.where(keep, v, pv)
            i = jnp.where(keep, i, pi)
            j //= 2
        k *= 2
    return v[:, :TOPK], i[:, :TOPK]


def _make_class_call(W, boff, nblocks):
    def kern(q2_ref, k_ref, w_ref, vals_ref, idx_ref):
        b = pl.program_id(0)
        q2 = q2_ref[...].reshape(RB * NH, DH)
        scores = jax.lax.dot_general(
            q2, k_ref[...], (((1,), (1,)), ((), ())),
            preferred_element_type=jnp.float32)
        s2 = jnp.maximum(scores, 0.0)
        w = w_ref[...]
        wtile = jnp.broadcast_to(w[:, None, :], (RB, RB, NH)).reshape(RB, RB * NH)
        rowt = jax.lax.broadcasted_iota(jnp.int32, (RB, RB * NH), 0)
        colt = jax.lax.broadcasted_iota(jnp.int32, (RB, RB * NH), 1) // NH
        W2 = jnp.where(rowt == colt, wtile, 0.0)
        lg = jnp.dot(W2, s2, preferred_element_type=jnp.float32)
        row = (boff + b) * RB + jax.lax.broadcasted_iota(jnp.int32, (RB, W), 0)
        col = jax.lax.broadcasted_iota(jnp.int32, (RB, W), 1)
        lg = jnp.where(row >= col, lg, jnp.float32(-1e30))
        vals, idx = _topk_sort(lg)
        vals_ref[...] = vals
        idx_ref[...] = idx

    return pl.pallas_call(
        kern,
        grid=(nblocks,),
        in_specs=[pl.BlockSpec((RB, NH * DH), lambda b: (b + boff, 0)),
                  pl.BlockSpec((W, DH), lambda b: (0, 0)),
                  pl.BlockSpec((RB, NH), lambda b: (b + boff, 0))],
        out_specs=[pl.BlockSpec((RB, TOPK), lambda b: (b, 0)),
                   pl.BlockSpec((RB, TOPK), lambda b: (b, 0))],
        out_shape=[jax.ShapeDtypeStruct((nblocks * RB, TOPK), jnp.float32),
                   jax.ShapeDtypeStruct((nblocks * RB, TOPK), jnp.int32)],
    )


def kernel(hidden_states, q_lora, positions, wq_b, wk, k_norm_w, k_norm_b,
           weights_proj, cos_sin_cache):
    rot_dim = DH - ROPE_HD
    q = (q_lora @ wq_b).reshape(T, NH, DH)
    k = _layernorm(hidden_states @ wk, k_norm_w, k_norm_b, EPS)
    cos = cos_sin_cache[positions, : rot_dim // 2]
    sin = cos_sin_cache[positions, rot_dim // 2:]
    q_pe = _apply_rope(q[:, :, :rot_dim], cos[:, None, :], sin[:, None, :])
    q = jnp.concatenate([q_pe, q[:, :, rot_dim:]], axis=-1)
    k_pe = _apply_rope(k[:, :rot_dim], cos, sin)
    k = jnp.concatenate([k_pe, k[:, rot_dim:]], axis=-1)
    q = q @ _HAD
    k = k @ _HAD
    w = (hidden_states @ weights_proj) * SOFTMAX_SCALE * WEIGHTS_SCALE

    q2 = q.reshape(T, NH * DH)
    v0, i0 = _make_class_call(512, 0, 4)(q2, k, w)
    v1, i1 = _make_class_call(1024, 4, 4)(q2, k, w)
    v2, i2 = _make_class_call(2048, 8, 8)(q2, k, w)
    vals = jnp.concatenate([v0, v1, v2], axis=0)
    idx = jnp.concatenate([i0, i1, i2], axis=0)
    return vals, idx
